# baseline (speedup 1.0000x reference)
import math

import jax
import jax.numpy as jnp
from jax.experimental import pallas as pl
from jax.experimental.pallas import tpu as pltpu

_P_DROP = 0.5
_BLK = 256
_NCHUNK = 10
_CH = 1000


def _prologue_body(x_ref, keep_ref, beta_ref, xnl_ref, xnT_ref, xdaT_ref,
                   eself_ref):
    x = x_ref[...]
    keep = keep_ref[...]
    d = x.shape[1]
    xd = jnp.where(keep != 0.0, x * (1.0 / (1.0 - _P_DROP)), 0.0)
    norm = jnp.sqrt(jnp.sum(xd * xd, axis=1, keepdims=True))
    xn = xd / jnp.maximum(norm, 1e-12)
    beta = beta_ref[0, 0]
    xnl_ref[...] = (xn * (beta * math.log2(math.e))).astype(jnp.bfloat16)
    xnT = jnp.transpose(xn)
    xnT_ref[...] = xnT.astype(jnp.bfloat16)
    xdaT_ref[0:d, :] = jnp.transpose(xd.astype(jnp.bfloat16))
    rsub = jax.lax.broadcasted_iota(jnp.int32, (8, x.shape[0]), 0)
    xdaT_ref[d:d + 8, :] = jnp.where(rsub == 0, 1.0, 0.0).astype(jnp.bfloat16)
    nsq = jnp.sum(xnT * xnT, axis=0, keepdims=True)
    eself_ref[...] = jnp.exp(beta * nsq)


def _main_body(xnl_ref, xnT_ref, xdaT_ref, xdaTb_ref, eself_ref, adj_hbm,
               outT_ref, buf, sems):
    dd, blk = outT_ref.shape
    n = adj_hbm.shape[0]
    nfull = n // _BLK
    nwidth = ((n - nfull * _BLK + 127) // 128) * 128
    j = pl.program_id(0)

    def descr(block_idx, ck, width):
        slot = jax.lax.rem(block_idx, 2)
        cs = block_idx * _BLK
        return pltpu.make_async_copy(
            adj_hbm.at[pl.ds(ck * _CH, _CH), pl.ds(cs, width)],
            buf.at[slot, pl.ds(ck * _CH, _CH), pl.ds(0, width)],
            sems.at[slot, ck],
        )

    def issue(block_idx, width):
        for ck in range(_NCHUNK):
            descr(block_idx, ck, width).start()

    @pl.when(j == 0)
    def _():
        issue(0, _BLK)

    nxt = j + 1

    @pl.when(nxt < nfull)
    def _():
        issue(nxt, _BLK)

    @pl.when(nxt == nfull)
    def _():
        issue(nxt, nwidth)

    @pl.when(j < nfull)
    def _():
        for ck in range(_NCHUNK):
            descr(j, ck, _BLK).wait()

    @pl.when(j == nfull)
    def _():
        for ck in range(_NCHUNK):
            descr(j, ck, nwidth).wait()

    slot = jax.lax.rem(j, 2)
    adj_blk = buf[slot, 0:n, :]
    s2 = jax.lax.dot_general(
        xnl_ref[...], xnT_ref[...], (((1,), (0,)), ((), ())),
        preferred_element_type=jnp.float32)
    exm = (jnp.exp2(s2) * adj_blk).astype(jnp.bfloat16)
    acc = jax.lax.dot_general(
        xdaT_ref[:, 0:n], exm, (((1,), (0,)), ((), ())),
        preferred_element_type=jnp.float32)
    num = acc[0:dd, :]
    den = acc[dd:dd + 1, :]
    adj_sub = buf[slot, pl.ds(j * blk, blk), :]
    rs = jax.lax.broadcasted_iota(jnp.int32, (blk, blk), 0)
    cs_i = jax.lax.broadcasted_iota(jnp.int32, (blk, blk), 1)
    adj_diag = jnp.sum(jnp.where(rs == cs_i, adj_sub, 0.0),
                       axis=0, keepdims=True)
    w = eself_ref[...] * (1.0 - adj_diag)
    xdTb = xdaTb_ref[0:dd, :].astype(jnp.float32)
    outT_ref[...] = (num + xdTb * w) * (1.0 / (den + w + 1e-16))


def _epilogue_body(outT_ref, out_ref):
    n, d = out_ref.shape
    t = outT_ref[:, 0:n]
    mu = jnp.mean(t, axis=1, keepdims=True)
    t = t - mu
    rn = jnp.sum(t * t, axis=0, keepdims=True)
    t = t * (1.0 / jnp.sqrt(1e-6 + rn))
    out_ref[...] = jnp.transpose(jnp.maximum(t, 0.0))


def kernel(x, adj, beta):
    n, d = x.shape
    npad = ((n + _BLK - 1) // _BLK) * _BLK
    dk = jax.random.fold_in(jax.random.key(42), 1)
    keep = jax.random.bernoulli(dk, 1.0 - _P_DROP, x.shape).astype(jnp.float32)
    beta_arr = jnp.reshape(beta.astype(jnp.float32), (1, 1))

    xnl, xnT, xdaT, eself = pl.pallas_call(
        _prologue_body,
        in_specs=[
            pl.BlockSpec((n, d), lambda: (0, 0)),
            pl.BlockSpec((n, d), lambda: (0, 0)),
            pl.BlockSpec(memory_space=pltpu.SMEM),
        ],
        out_specs=[
            pl.BlockSpec((n, d), lambda: (0, 0)),
            pl.BlockSpec((d, n), lambda: (0, 0)),
            pl.BlockSpec((d + 8, n), lambda: (0, 0)),
            pl.BlockSpec((1, n), lambda: (0, 0)),
        ],
        out_shape=[
            jax.ShapeDtypeStruct((n, d), jnp.bfloat16),
            jax.ShapeDtypeStruct((d, n), jnp.bfloat16),
            jax.ShapeDtypeStruct((d + 8, n), jnp.bfloat16),
            jax.ShapeDtypeStruct((1, n), jnp.float32),
        ],
    )(x, keep, beta_arr)

    pad = npad - n
    xnT = jnp.pad(xnT, ((0, 0), (0, pad)))
    xdaT = jnp.pad(xdaT, ((0, 0), (0, pad)))
    eself = jnp.pad(eself, ((0, 0), (0, pad)))

    grid = npad // _BLK
    outT = pl.pallas_call(
        _main_body,
        grid=(grid,),
        in_specs=[
            pl.BlockSpec((n, d), lambda j: (0, 0)),
            pl.BlockSpec((d, _BLK), lambda j: (0, j)),
            pl.BlockSpec((d + 8, npad), lambda j: (0, 0)),
            pl.BlockSpec((d + 8, _BLK), lambda j: (0, j)),
            pl.BlockSpec((1, _BLK), lambda j: (0, j)),
            pl.BlockSpec(memory_space=pl.ANY),
        ],
        out_specs=pl.BlockSpec((d, _BLK), lambda j: (0, j)),
        out_shape=jax.ShapeDtypeStruct((d, npad), jnp.float32),
        scratch_shapes=[
            pltpu.VMEM((2, npad, _BLK), jnp.float32),
            pltpu.SemaphoreType.DMA((2, _NCHUNK)),
        ],
        compiler_params=pltpu.CompilerParams(
            dimension_semantics=("arbitrary",),
        ),
    )(xnl, xnT, xdaT, xdaT, eself, adj)

    out = pl.pallas_call(
        _epilogue_body,
        in_specs=[pl.BlockSpec((d, npad), lambda: (0, 0))],
        out_specs=pl.BlockSpec((n, d), lambda: (0, 0)),
        out_shape=jax.ShapeDtypeStruct((n, d), jnp.float32),
    )(outT)

    return (out, adj)


# speedup vs baseline: 1.0277x; 1.0277x over previous
import math

import jax
import jax.numpy as jnp
from jax.experimental import pallas as pl
from jax.experimental.pallas import tpu as pltpu

_P_DROP = 0.5
_BLK = 256


def _prologue_body(x_ref, keep_ref, beta_ref, xnl_ref, xnT_ref, xdaT_ref,
                   eself_ref):
    x = x_ref[...]
    keep = keep_ref[...]
    d = x.shape[1]
    xd = jnp.where(keep != 0.0, x * (1.0 / (1.0 - _P_DROP)), 0.0)
    norm = jnp.sqrt(jnp.sum(xd * xd, axis=1, keepdims=True))
    xn = xd / jnp.maximum(norm, 1e-12)
    beta = beta_ref[0, 0]
    xnl_ref[...] = (xn * (beta * math.log2(math.e))).astype(jnp.bfloat16)
    xnT = jnp.transpose(xn)
    xnT_ref[...] = xnT.astype(jnp.bfloat16)
    xdaT_ref[0:d, :] = jnp.transpose(xd.astype(jnp.bfloat16))
    rsub = jax.lax.broadcasted_iota(jnp.int32, (8, x.shape[0]), 0)
    xdaT_ref[d:d + 8, :] = jnp.where(rsub == 0, 1.0, 0.0).astype(jnp.bfloat16)
    nsq = jnp.sum(xnT * xnT, axis=0, keepdims=True)
    eself_ref[...] = jnp.exp(beta * nsq)


def _main_body(xnl_ref, xnT_ref, xdaT_ref, xdaTb_ref, adj_ref, eself_ref,
               out_ref, outT):
    dd, blk = xdaTb_ref.shape[0] - 8, xdaTb_ref.shape[1]
    n = adj_ref.shape[0]
    s2 = jax.lax.dot_general(
        xnl_ref[...], xnT_ref[...], (((1,), (0,)), ((), ())),
        preferred_element_type=jnp.float32)
    exm = (jnp.exp2(s2) * adj_ref[...]).astype(jnp.bfloat16)
    acc = jax.lax.dot_general(
        xdaT_ref[...], exm, (((1,), (0,)), ((), ())),
        preferred_element_type=jnp.float32)
    num = acc[0:dd, :]
    den = acc[dd:dd + 1, :]
    j = pl.program_id(0)
    start = jnp.minimum(j * blk, n - blk)
    delta = j * blk - start
    adj_sub = adj_ref[pl.ds(start, blk), :]
    rs = jax.lax.broadcasted_iota(jnp.int32, (blk, blk), 0)
    cs = jax.lax.broadcasted_iota(jnp.int32, (blk, blk), 1)
    ident = jnp.where(rs == cs + delta, 1.0, 0.0)
    adj_diag = jnp.sum(adj_sub * ident, axis=0, keepdims=True)
    w = eself_ref[...] * (1.0 - adj_diag)
    xdTb = xdaTb_ref[0:dd, :].astype(jnp.float32)
    outT[:, pl.ds(j * blk, blk)] = (
        (num + xdTb * w) * (1.0 / (den + w + 1e-16)))

    @pl.when(j == pl.num_programs(0) - 1)
    def _():
        t = outT[:, 0:n]
        mu = jnp.mean(t, axis=1, keepdims=True)
        t = t - mu
        rn = jnp.sum(t * t, axis=0, keepdims=True)
        t = t * (1.0 / jnp.sqrt(1e-6 + rn))
        out_ref[...] = jnp.transpose(jnp.maximum(t, 0.0))


def kernel(x, adj, beta):
    n, d = x.shape
    dk = jax.random.fold_in(jax.random.key(42), 1)
    keep = jax.random.bernoulli(dk, 1.0 - _P_DROP, x.shape).astype(jnp.float32)
    beta_arr = jnp.reshape(beta.astype(jnp.float32), (1, 1))

    xnl, xnT, xdaT, eself = pl.pallas_call(
        _prologue_body,
        in_specs=[
            pl.BlockSpec((n, d), lambda: (0, 0)),
            pl.BlockSpec((n, d), lambda: (0, 0)),
            pl.BlockSpec(memory_space=pltpu.SMEM),
        ],
        out_specs=[
            pl.BlockSpec((n, d), lambda: (0, 0)),
            pl.BlockSpec((d, n), lambda: (0, 0)),
            pl.BlockSpec((d + 8, n), lambda: (0, 0)),
            pl.BlockSpec((1, n), lambda: (0, 0)),
        ],
        out_shape=[
            jax.ShapeDtypeStruct((n, d), jnp.bfloat16),
            jax.ShapeDtypeStruct((d, n), jnp.bfloat16),
            jax.ShapeDtypeStruct((d + 8, n), jnp.bfloat16),
            jax.ShapeDtypeStruct((1, n), jnp.float32),
        ],
    )(x, keep, beta_arr)

    grid = pl.cdiv(n, _BLK)
    out = pl.pallas_call(
        _main_body,
        grid=(grid,),
        in_specs=[
            pl.BlockSpec((n, d), lambda j: (0, 0)),
            pl.BlockSpec((d, _BLK), lambda j: (0, j)),
            pl.BlockSpec((d + 8, n), lambda j: (0, 0)),
            pl.BlockSpec((d + 8, _BLK), lambda j: (0, j)),
            pl.BlockSpec((n, _BLK), lambda j: (0, j)),
            pl.BlockSpec((1, _BLK), lambda j: (0, j)),
        ],
        out_specs=pl.BlockSpec((n, d), lambda j: (0, 0)),
        out_shape=jax.ShapeDtypeStruct((n, d), jnp.float32),
        scratch_shapes=[
            pltpu.VMEM((d, grid * _BLK), jnp.float32),
        ],
        compiler_params=pltpu.CompilerParams(
            dimension_semantics=("arbitrary",),
        ),
    )(xnl, xnT, xdaT, xdaT, adj, eself)

    return (out, adj)
